# Initial kernel scaffold; baseline (speedup 1.0000x reference)
import jax
import jax.numpy as jnp
from jax.experimental import pallas as pl


def kernel(x, edge_index, W1, b1, W2, b2, Wc, bc):
    raise NotImplementedError("write your pallas kernel here")


# speedup vs baseline: 10.6481x; 10.6481x over previous
import functools

import jax
import jax.numpy as jnp
from jax import lax
from jax.experimental import pallas as pl
from jax.experimental.pallas import tpu as pltpu
from jax.experimental.pallas import tpu_sc as plsc

N = 10000
E = 160000
D = 256
H = 128
NC = 2
NS = 16
ROWS_PER_TILE = N // NS
EW = 80
ER = E // EW
ERT = ER // NS
DW = 125
DR = E // DW
DRT = DR // (NC * NS)
DEGW = 128

_f32 = jnp.float32
_i32 = jnp.int32


def _mesh():
    return plsc.VectorSubcoreMesh(
        core_axis_name="c", subcore_axis_name="s",
        num_cores=NC, num_subcores=NS)


def _deg_body(dst_hbm, ones_hbm, zeros_hbm, out_hbm, ones_v, idx_v, acc_sp):
    c = lax.axis_index("c")
    s = lax.axis_index("s")
    pltpu.sync_copy(ones_hbm, ones_v)
    pltpu.sync_copy(dst_hbm.at[c * NS + s], idx_v)
    pltpu.sync_copy(zeros_hbm, acc_sp.at[pl.ds(s * ROWS_PER_TILE, ROWS_PER_TILE)])
    plsc.subcore_barrier()

    def body(j, carry):
        pltpu.sync_copy(ones_v, acc_sp.at[idx_v.at[j]], add=True)
        return carry

    lax.fori_loop(0, DRT, body, 0)
    plsc.subcore_barrier()
    pltpu.sync_copy(
        acc_sp.at[pl.ds(s * ROWS_PER_TILE, ROWS_PER_TILE)],
        out_hbm.at[c, s],
    )


def _edge_body(yl_hbm, yr_hbm, src_hbm, dst_hbm, zeros_hbm,
               outl_hbm, outr_hbm, sidx, didx, rows, acc_sp, sem):
    c = lax.axis_index("c")
    s = lax.axis_index("s")
    pltpu.sync_copy(src_hbm.at[s], sidx)
    pltpu.sync_copy(dst_hbm.at[s], didx)

    def work(y_hbm, out_hbm):
        pltpu.sync_copy(
            zeros_hbm, acc_sp.at[pl.ds(s * ROWS_PER_TILE, ROWS_PER_TILE)])
        plsc.subcore_barrier()

        def body(j, carry):
            pltpu.async_copy(y_hbm.at[sidx.at[j]], rows, sem).wait()
            pltpu.sync_copy(rows, acc_sp.at[didx.at[j]], add=True)
            return carry

        lax.fori_loop(0, ERT, body, 0)
        plsc.subcore_barrier()
        pltpu.sync_copy(
            acc_sp.at[pl.ds(s * ROWS_PER_TILE, ROWS_PER_TILE)],
            out_hbm.at[s],
        )

    pl.when(c == 0)(lambda: work(yl_hbm, outl_hbm))
    pl.when(c == 1)(lambda: work(yr_hbm, outr_hbm))


_SC_KERNELS = None


def _sc_kernels():
    global _SC_KERNELS
    if _SC_KERNELS is None:
        deg_pass = pl.kernel(
            _deg_body,
            out_type=jax.ShapeDtypeStruct((NC, NS, ROWS_PER_TILE, DEGW), _f32),
            mesh=_mesh(),
            scratch_types=[
                pltpu.VMEM((DW, DEGW), _f32),
                pltpu.VMEM((DRT, DW), _i32),
                pltpu.VMEM_SHARED((N, DEGW), _f32),
            ],
        )
        edge_pass = pl.kernel(
            _edge_body,
            out_type=(
                jax.ShapeDtypeStruct((NS, ROWS_PER_TILE, H), _f32),
                jax.ShapeDtypeStruct((NS, ROWS_PER_TILE, H), _f32),
            ),
            mesh=_mesh(),
            scratch_types=[
                pltpu.VMEM((ERT, EW), _i32),
                pltpu.VMEM((ERT, EW), _i32),
                pltpu.VMEM((EW, H), _f32),
                pltpu.VMEM_SHARED((N, H), _f32),
                pltpu.SemaphoreType.DMA,
            ],
        )
        _SC_KERNELS = (deg_pass, edge_pass)
    return _SC_KERNELS


_BLK = 1000
_GRID = N // _BLK


def _dinv_of(degp_blk):
    deg = degp_blk[0, :, 0] + degp_blk[1, :, 0] + 1.0
    return lax.rsqrt(deg)


def _k2_body(x_ref, w1_ref, b1_ref, degp_ref, yl_ref, yr_ref, st_ref):
    xw = jnp.dot(x_ref[...], w1_ref[...], preferred_element_type=_f32)
    dinv = _dinv_of(degp_ref[...])[:, None]
    y = xw * dinv
    yl_ref[...] = y[:, :H]
    yr_ref[...] = y[:, H:]
    st_ref[...] = y * dinv + b1_ref[...]


def _k4_body(accl_ref, accr_ref, st_ref, degp_ref, w2_ref, b2_ref,
             yl_ref, yr_ref, st2_ref):
    dinv = _dinv_of(degp_ref[...])[:, None]
    acc = jnp.concatenate([accl_ref[...], accr_ref[...]], axis=1)
    h1 = jnp.maximum(acc * dinv + st_ref[...], 0.0)
    xw2 = jnp.dot(h1, w2_ref[...], preferred_element_type=_f32)
    y2 = xw2 * dinv
    yl_ref[...] = y2[:, :H]
    yr_ref[...] = y2[:, H:]
    st2_ref[...] = y2 * dinv + b2_ref[...]


def _k6_body(accl_ref, accr_ref, st2_ref, degp_ref, wc_ref, bc_ref,
             out_ref, sum_ref):
    i = pl.program_id(0)

    @pl.when(i == 0)
    def _init():
        sum_ref[...] = jnp.zeros_like(sum_ref)

    dinv = _dinv_of(degp_ref[...])[:, None]
    acc = jnp.concatenate([accl_ref[...], accr_ref[...]], axis=1)
    h2 = jnp.maximum(acc * dinv + st2_ref[...], 0.0)
    sum_ref[...] += jnp.sum(h2, axis=0, keepdims=True)

    @pl.when(i == _GRID - 1)
    def _fin():
        mean = sum_ref[...] * (1.0 / N)
        logits = jnp.dot(mean, wc_ref[...],
                         preferred_element_type=_f32) + bc_ref[...]
        mx = jnp.max(logits, axis=1, keepdims=True)
        z = logits - mx
        out_ref[...] = z - jnp.log(jnp.sum(jnp.exp(z), axis=1, keepdims=True))


def _row_spec(w):
    return pl.BlockSpec((_BLK, w), lambda i: (i, 0))


_DEGP_SPEC = pl.BlockSpec((NC, _BLK, DEGW), lambda i: (0, i, 0))
_FULL2 = lambda a, b: pl.BlockSpec((a, b), lambda i: (0, 0))


def _k2(x, W1, b1r, degp):
    return pl.pallas_call(
        _k2_body,
        grid=(_GRID,),
        in_specs=[_row_spec(D), _FULL2(D, D), _FULL2(1, D), _DEGP_SPEC],
        out_specs=[_row_spec(H), _row_spec(H), _row_spec(D)],
        out_shape=[
            jax.ShapeDtypeStruct((N, H), _f32),
            jax.ShapeDtypeStruct((N, H), _f32),
            jax.ShapeDtypeStruct((N, D), _f32),
        ],
    )(x, W1, b1r, degp)


def _k4(accl, accr, st1, degp, W2, b2r):
    return pl.pallas_call(
        _k4_body,
        grid=(_GRID,),
        in_specs=[_row_spec(H), _row_spec(H), _row_spec(D), _DEGP_SPEC,
                  _FULL2(D, D), _FULL2(1, D)],
        out_specs=[_row_spec(H), _row_spec(H), _row_spec(D)],
        out_shape=[
            jax.ShapeDtypeStruct((N, H), _f32),
            jax.ShapeDtypeStruct((N, H), _f32),
            jax.ShapeDtypeStruct((N, D), _f32),
        ],
    )(accl, accr, st1, degp, W2, b2r)


def _k6(accl, accr, st2, degp, Wc, bcr):
    return pl.pallas_call(
        _k6_body,
        grid=(_GRID,),
        in_specs=[_row_spec(H), _row_spec(H), _row_spec(D), _DEGP_SPEC,
                  _FULL2(D, 2), _FULL2(1, 2)],
        out_specs=pl.BlockSpec((1, 2), lambda i: (0, 0)),
        out_shape=jax.ShapeDtypeStruct((1, 2), _f32),
        scratch_shapes=[pltpu.VMEM((1, D), _f32)],
    )(accl, accr, st2, degp, Wc, bcr)


@jax.jit
def kernel(x, edge_index, W1, b1, W2, b2, Wc, bc):
    src = edge_index[0].reshape(NS, ERT, EW)
    dst = edge_index[1].reshape(NS, ERT, EW)
    dst40 = edge_index[1].reshape(NC * NS, DRT, DW)

    ones_deg = jnp.ones((DW, DEGW), _f32)
    zeros_h = jnp.zeros((ROWS_PER_TILE, H), _f32)

    deg_pass, edge_pass = _sc_kernels()
    degp = deg_pass(dst40, ones_deg, zeros_h).reshape(NC, N, DEGW)

    yl, yr, st1 = _k2(x, W1, b1.reshape(1, D), degp)
    accl, accr = edge_pass(yl, yr, src, dst, zeros_h)
    yl2, yr2, st2 = _k4(accl.reshape(N, H), accr.reshape(N, H), st1, degp,
                        W2, b2.reshape(1, D))
    a2l, a2r = edge_pass(yl2, yr2, src, dst, zeros_h)
    return _k6(a2l.reshape(N, H), a2r.reshape(N, H), st2, degp,
               Wc, bc.reshape(1, 2))


# speedup vs baseline: 11.0546x; 1.0382x over previous
import functools

import jax
import jax.numpy as jnp
from jax import lax
from jax.experimental import pallas as pl
from jax.experimental.pallas import tpu as pltpu
from jax.experimental.pallas import tpu_sc as plsc

N = 10000
E = 160000
D = 256
H = 128
NC = 2
NS = 16
ROWS_PER_TILE = N // NS
EW = 80
ER = E // EW
ERT = ER // NS
EPT = ERT * EW
DW = 125
DR = E // DW
DRT = DR // (NC * NS)
DEGW = 128

_f32 = jnp.float32
_i32 = jnp.int32


def _mesh():
    return plsc.VectorSubcoreMesh(
        core_axis_name="c", subcore_axis_name="s",
        num_cores=NC, num_subcores=NS)


def _deg_body(dst_hbm, ones_hbm, zeros_hbm, out_hbm, ones_v, idx_v, acc_sp):
    c = lax.axis_index("c")
    s = lax.axis_index("s")
    pltpu.sync_copy(ones_hbm, ones_v)
    pltpu.sync_copy(dst_hbm.at[c * NS + s], idx_v)
    pltpu.sync_copy(zeros_hbm, acc_sp.at[pl.ds(s * ROWS_PER_TILE, ROWS_PER_TILE)])
    plsc.subcore_barrier()

    def body(j, carry):
        pltpu.sync_copy(ones_v, acc_sp.at[idx_v.at[j]], add=True)
        return carry

    lax.fori_loop(0, DRT, body, 0)
    plsc.subcore_barrier()
    pltpu.sync_copy(
        acc_sp.at[pl.ds(s * ROWS_PER_TILE, ROWS_PER_TILE)],
        out_hbm.at[c, s],
    )


def _edge_body(yl_hbm, yr_hbm, src_hbm, dst_hbm, zeros_hbm,
               outl_hbm, outr_hbm, sidx, didx, rows0, rows1, acc_sp,
               sg0, sg1, ss0, ss1):
    c = lax.axis_index("c")
    s = lax.axis_index("s")
    pltpu.sync_copy(src_hbm.at[s], sidx)
    pltpu.sync_copy(dst_hbm.at[s], didx)

    def work(y_hbm, out_hbm):
        pltpu.sync_copy(
            zeros_hbm, acc_sp.at[pl.ds(s * ROWS_PER_TILE, ROWS_PER_TILE)])
        plsc.subcore_barrier()

        def gather(j, buf, sem):
            pltpu.async_copy(y_hbm.at[sidx.at[pl.ds(j * EW, EW)]], buf, sem)

        def gwait(j, buf, sem):
            pltpu.make_async_copy(
                y_hbm.at[sidx.at[pl.ds(j * EW, EW)]], buf, sem).wait()

        def scatter(j, buf, sem):
            pltpu.async_copy(buf, acc_sp.at[didx.at[j]], sem, add=True)

        def swait(j, buf, sem):
            pltpu.make_async_copy(buf, acc_sp.at[didx.at[j]], sem).wait()

        gather(0, rows0, sg0)
        gwait(0, rows0, sg0)
        gather(1, rows1, sg1)
        scatter(0, rows0, ss0)

        def body(i, carry):
            j = 2 * i + 1
            gwait(j, rows1, sg1)
            swait(j - 1, rows0, ss0)
            gather(j + 1, rows0, sg0)
            scatter(j, rows1, ss1)
            gwait(j + 1, rows0, sg0)
            swait(j, rows1, ss1)
            gather(j + 2, rows1, sg1)
            scatter(j + 1, rows0, ss0)
            return carry

        lax.fori_loop(0, (ERT - 1) // 2, body, 0)
        swait(ERT - 1, rows0, ss0)
        gwait(ERT, rows1, sg1)

        plsc.subcore_barrier()
        pltpu.sync_copy(
            acc_sp.at[pl.ds(s * ROWS_PER_TILE, ROWS_PER_TILE)],
            out_hbm.at[s],
        )

    pl.when(c == 0)(lambda: work(yl_hbm, outl_hbm))
    pl.when(c == 1)(lambda: work(yr_hbm, outr_hbm))


_SC_KERNELS = None


def _sc_kernels():
    global _SC_KERNELS
    if _SC_KERNELS is None:
        deg_pass = pl.kernel(
            _deg_body,
            out_type=jax.ShapeDtypeStruct((NC, NS, ROWS_PER_TILE, DEGW), _f32),
            mesh=_mesh(),
            scratch_types=[
                pltpu.VMEM((DW, DEGW), _f32),
                pltpu.VMEM((DRT, DW), _i32),
                pltpu.VMEM_SHARED((N, DEGW), _f32),
            ],
        )
        edge_pass = pl.kernel(
            _edge_body,
            out_type=(
                jax.ShapeDtypeStruct((NS, ROWS_PER_TILE, H), _f32),
                jax.ShapeDtypeStruct((NS, ROWS_PER_TILE, H), _f32),
            ),
            mesh=_mesh(),
            scratch_types=[
                pltpu.VMEM((EPT + EW,), _i32),
                pltpu.VMEM((ERT, EW), _i32),
                pltpu.VMEM((EW, H), _f32),
                pltpu.VMEM((EW, H), _f32),
                pltpu.VMEM_SHARED((N, H), _f32),
                pltpu.SemaphoreType.DMA,
                pltpu.SemaphoreType.DMA,
                pltpu.SemaphoreType.DMA,
                pltpu.SemaphoreType.DMA,
            ],
        )
        _SC_KERNELS = (deg_pass, edge_pass)
    return _SC_KERNELS


_BLK = 1000
_GRID = N // _BLK


def _dinv_of(degp_blk):
    deg = degp_blk[0, :, 0] + degp_blk[1, :, 0] + 1.0
    return lax.rsqrt(deg)


def _k2_body(x_ref, w1_ref, b1_ref, degp_ref, yl_ref, yr_ref, st_ref):
    xw = jnp.dot(x_ref[...], w1_ref[...], preferred_element_type=_f32)
    dinv = _dinv_of(degp_ref[...])[:, None]
    y = xw * dinv
    yl_ref[...] = y[:, :H]
    yr_ref[...] = y[:, H:]
    st_ref[...] = y * dinv + b1_ref[...]


def _k4_body(accl_ref, accr_ref, st_ref, degp_ref, w2_ref, b2_ref,
             yl_ref, yr_ref, st2_ref):
    dinv = _dinv_of(degp_ref[...])[:, None]
    acc = jnp.concatenate([accl_ref[...], accr_ref[...]], axis=1)
    h1 = jnp.maximum(acc * dinv + st_ref[...], 0.0)
    xw2 = jnp.dot(h1, w2_ref[...], preferred_element_type=_f32)
    y2 = xw2 * dinv
    yl_ref[...] = y2[:, :H]
    yr_ref[...] = y2[:, H:]
    st2_ref[...] = y2 * dinv + b2_ref[...]


def _k6_body(accl_ref, accr_ref, st2_ref, degp_ref, wc_ref, bc_ref,
             out_ref, sum_ref):
    i = pl.program_id(0)

    @pl.when(i == 0)
    def _init():
        sum_ref[...] = jnp.zeros_like(sum_ref)

    dinv = _dinv_of(degp_ref[...])[:, None]
    acc = jnp.concatenate([accl_ref[...], accr_ref[...]], axis=1)
    h2 = jnp.maximum(acc * dinv + st2_ref[...], 0.0)
    sum_ref[...] += jnp.sum(h2, axis=0, keepdims=True)

    @pl.when(i == _GRID - 1)
    def _fin():
        mean = sum_ref[...] * (1.0 / N)
        logits = jnp.dot(mean, wc_ref[...],
                         preferred_element_type=_f32) + bc_ref[...]
        mx = jnp.max(logits, axis=1, keepdims=True)
        z = logits - mx
        out_ref[...] = z - jnp.log(jnp.sum(jnp.exp(z), axis=1, keepdims=True))


def _row_spec(w):
    return pl.BlockSpec((_BLK, w), lambda i: (i, 0))


_DEGP_SPEC = pl.BlockSpec((NC, _BLK, DEGW), lambda i: (0, i, 0))
_FULL2 = lambda a, b: pl.BlockSpec((a, b), lambda i: (0, 0))


def _k2(x, W1, b1r, degp):
    return pl.pallas_call(
        _k2_body,
        grid=(_GRID,),
        in_specs=[_row_spec(D), _FULL2(D, D), _FULL2(1, D), _DEGP_SPEC],
        out_specs=[_row_spec(H), _row_spec(H), _row_spec(D)],
        out_shape=[
            jax.ShapeDtypeStruct((N, H), _f32),
            jax.ShapeDtypeStruct((N, H), _f32),
            jax.ShapeDtypeStruct((N, D), _f32),
        ],
    )(x, W1, b1r, degp)


def _k4(accl, accr, st1, degp, W2, b2r):
    return pl.pallas_call(
        _k4_body,
        grid=(_GRID,),
        in_specs=[_row_spec(H), _row_spec(H), _row_spec(D), _DEGP_SPEC,
                  _FULL2(D, D), _FULL2(1, D)],
        out_specs=[_row_spec(H), _row_spec(H), _row_spec(D)],
        out_shape=[
            jax.ShapeDtypeStruct((N, H), _f32),
            jax.ShapeDtypeStruct((N, H), _f32),
            jax.ShapeDtypeStruct((N, D), _f32),
        ],
    )(accl, accr, st1, degp, W2, b2r)


def _k6(accl, accr, st2, degp, Wc, bcr):
    return pl.pallas_call(
        _k6_body,
        grid=(_GRID,),
        in_specs=[_row_spec(H), _row_spec(H), _row_spec(D), _DEGP_SPEC,
                  _FULL2(D, 2), _FULL2(1, 2)],
        out_specs=pl.BlockSpec((1, 2), lambda i: (0, 0)),
        out_shape=jax.ShapeDtypeStruct((1, 2), _f32),
        scratch_shapes=[pltpu.VMEM((1, D), _f32)],
    )(accl, accr, st2, degp, Wc, bcr)


@jax.jit
def kernel(x, edge_index, W1, b1, W2, b2, Wc, bc):
    src = jnp.pad(edge_index[0].reshape(NS, EPT), ((0, 0), (0, EW)))
    dst = edge_index[1].reshape(NS, ERT, EW)
    dst40 = edge_index[1].reshape(NC * NS, DRT, DW)

    ones_deg = jnp.ones((DW, DEGW), _f32)
    zeros_h = jnp.zeros((ROWS_PER_TILE, H), _f32)

    deg_pass, edge_pass = _sc_kernels()
    degp = deg_pass(dst40, ones_deg, zeros_h).reshape(NC, N, DEGW)

    yl, yr, st1 = _k2(x, W1, b1.reshape(1, D), degp)
    accl, accr = edge_pass(yl, yr, src, dst, zeros_h)
    yl2, yr2, st2 = _k4(accl.reshape(N, H), accr.reshape(N, H), st1, degp,
                        W2, b2.reshape(1, D))
    a2l, a2r = edge_pass(yl2, yr2, src, dst, zeros_h)
    return _k6(a2l.reshape(N, H), a2r.reshape(N, H), st2, degp,
               Wc, bc.reshape(1, 2))
